# Initial kernel scaffold; baseline (speedup 1.0000x reference)
import jax
import jax.numpy as jnp
from jax.experimental import pallas as pl


def kernel(inputs, edge_index, emb0, emb1, emb2, W1, al1, ar1, b1, W2, al2, ar2, b2):
    raise NotImplementedError("write your pallas kernel here")


# speedup vs baseline: 26.5478x; 26.5478x over previous
import functools

import jax
import jax.numpy as jnp
from jax import lax
from jax.experimental import pallas as pl
from jax.experimental.pallas import tpu as pltpu
from jax.experimental.pallas import tpu_sc as plsc

_N = 100000
_E = 1600000
_NBK = 1000
_NC = 2
_NS = 16
_NW = _NC * _NS
_CB = 128
_EPW = _E // _NW
_NFULL = _EPW // _CB
_REM = _EPW - _NFULL * _CB
_NA = 100096
_ZST = _NA // _NS


def _make_edge_pass(with_msg):
    mesh = plsc.VectorSubcoreMesh(
        core_axis_name="c", subcore_axis_name="s", num_cores=_NC)

    scratch = [
        pltpu.VMEM((_CB,), jnp.int32),
        pltpu.VMEM((_CB,), jnp.int32),
        pltpu.VMEM((_CB, 16), jnp.float32),
        pltpu.VMEM((_CB, 16), jnp.float32),
        pltpu.VMEM((_CB, 16), jnp.float32),
        pltpu.VMEM((16,), jnp.float32),
        pltpu.VMEM_SHARED((_NA, 16), jnp.float32),
        pltpu.SemaphoreType.DMA,
    ]
    if with_msg:
        scratch.insert(4, pltpu.VMEM((_CB, 16), jnp.float32))

    @functools.partial(
        pl.kernel, mesh=mesh,
        out_type=jax.ShapeDtypeStruct((_NC, _NA, 16), jnp.float32),
        scratch_types=scratch,
        compiler_params=pltpu.CompilerParams(use_tc_tiling_on_sc=False))
    def k(*refs):
        if with_msg:
            (src_h, dst_h, tabA_h, tabB_h, tabH_h, zeros_h, c_h, out_h,
             idxs, idxd, rA, rB, rH, msg, cv, accum, sem) = refs
        else:
            (src_h, dst_h, tabA_h, tabB_h, zeros_h, c_h, out_h,
             idxs, idxd, rA, rB, msg, cv, accum, sem) = refs

        cid = lax.axis_index("c")
        sid = lax.axis_index("s")
        wid = sid * _NC + cid

        pltpu.sync_copy(zeros_h.at[pl.ds(sid * _ZST, _ZST)],
                        accum.at[pl.ds(sid * _ZST, _ZST)])
        pltpu.sync_copy(c_h, cv)
        plsc.subcore_barrier()

        def do_chunk(base, size):
            pltpu.sync_copy(src_h.at[pl.ds(base, size)], idxs.at[pl.ds(0, size)])
            pltpu.sync_copy(dst_h.at[pl.ds(base, size)], idxd.at[pl.ds(0, size)])
            if size < _CB:
                for kk in range(size, _CB, 16):
                    idxs[pl.ds(kk, 16)] = jnp.zeros((16,), jnp.int32)
                    idxd[pl.ds(kk, 16)] = jnp.full((16,), _N, jnp.int32)
            pltpu.async_copy(tabA_h.at[idxs], rA, sem).wait()
            pltpu.async_copy(tabB_h.at[idxd], rB, sem).wait()
            if with_msg:
                pltpu.async_copy(tabH_h.at[idxs], rH, sem).wait()
            cvv = cv[...]

            def body(j, carry):
                e = rA[j] + rB[j]
                e = jnp.where(e > 0.0, e, 0.2 * e)
                w = jnp.exp(e - cvv)
                if with_msg:
                    msg[j] = w * rH[j]
                else:
                    msg[j] = w
                return carry

            lax.fori_loop(0, _CB, body, 0)
            pltpu.sync_copy(msg, accum.at[idxd], add=True)

        ebase = wid * _EPW

        def gbody(g, carry):
            do_chunk(ebase + g * _CB, _CB)
            return carry

        lax.fori_loop(0, _NFULL, gbody, 0)
        if _REM:
            do_chunk(ebase + _NFULL * _CB, _REM)

        plsc.subcore_barrier()
        pltpu.sync_copy(accum.at[pl.ds(sid * _ZST, _ZST)],
                        out_h.at[cid, pl.ds(sid * _ZST, _ZST)])

    return k


_edge_denom = _make_edge_pass(with_msg=False)
_edge_msg = _make_edge_pass(with_msg=True)


def _full_spec(shape):
    nd = len(shape)
    return pl.BlockSpec(shape, lambda i: (0,) * nd)


def _tc1(inputs, emb0p, emb1p, emb2p, W1, al1f, ar1f):
    def body(inp, e0, e1, e2, w1, al, ar,
             hh0, hh1, hh2, eb0, eb1, eb2, rb0, rb1, rb2, elt, ert):
        x = inp[...]
        oh0 = (x[:, 0:1].astype(jnp.int32) == lax.broadcasted_iota(
            jnp.int32, (_NBK, 16), 1)).astype(jnp.float32)
        oh1 = (x[:, 1:2].astype(jnp.int32) == lax.broadcasted_iota(
            jnp.int32, (_NBK, 8), 1)).astype(jnp.float32)
        oh2 = (x[:, 2:3].astype(jnp.int32) == lax.broadcasted_iota(
            jnp.int32, (_NBK, 16), 1)).astype(jnp.float32)
        x0 = jnp.dot(oh0, e0[...], preferred_element_type=jnp.float32)
        x1 = jnp.dot(oh1, e1[...], preferred_element_type=jnp.float32)
        x2 = jnp.dot(oh2, e2[...], preferred_element_type=jnp.float32)
        xx = jnp.concatenate([x0, x1, x2, x[:, 3:]], axis=1)
        h = jnp.dot(xx, w1[...], preferred_element_type=jnp.float32)
        sel = (lax.broadcasted_iota(jnp.int32, (48, 3), 0) // 16 ==
               lax.broadcasted_iota(jnp.int32, (48, 3), 1)).astype(jnp.float32)
        el = jnp.dot(h * al[0:1, :], sel,
                     preferred_element_type=jnp.float32)
        er = jnp.dot(h * ar[0:1, :], sel,
                     preferred_element_type=jnp.float32)
        for hd, (hr, ebr, rbr) in enumerate(
                [(hh0, eb0, rb0), (hh1, eb1, rb1), (hh2, eb2, rb2)]):
            hr[...] = h[:, hd * 16:(hd + 1) * 16]
            ebr[...] = jnp.broadcast_to(el[:, hd:hd + 1], (_NBK, 16))
            rbr[...] = jnp.broadcast_to(er[:, hd:hd + 1], (_NBK, 16))
        z = jnp.zeros((_NBK, 13), jnp.float32)
        elt[...] = jnp.concatenate([el, z], axis=1)
        ert[...] = jnp.concatenate([er, z], axis=1)

    grid = _N // _NBK
    out16 = jax.ShapeDtypeStruct((_N, 16), jnp.float32)
    nspec = pl.BlockSpec((_NBK, 16), lambda i: (i, 0))
    return pl.pallas_call(
        body,
        grid=grid,
        in_specs=[
            pl.BlockSpec((_NBK, 18), lambda i: (i, 0)),
            _full_spec((16, 8)), _full_spec((8, 3)), _full_spec((16, 6)),
            _full_spec((32, 48)), _full_spec((8, 48)), _full_spec((8, 48)),
        ],
        out_specs=[nspec] * 11,
        out_shape=[out16] * 11,
    )(inputs, emb0p, emb1p, emb2p, W1, al1f, ar1f)


def _tc2(P, D, W2, b1p, al2p, ar2p):
    def body(pref, dref, w2, b1r, al, ar, h2o, eb2o, rb2o):
        p = pref[...]
        d = dref[...]
        dh = d[0] + d[1]
        dh = jnp.where(dh == 0.0, 1.0, dh)
        cols = []
        for hd in range(3):
            num = p[2 * hd] + p[2 * hd + 1]
            cols.append(num / dh[:, hd:hd + 1])
        hid = jnp.concatenate(cols, axis=1) + b1r[0:1, :]
        hid = jnp.where(hid > 0.0, hid, 0.01 * hid)
        h2 = jnp.dot(hid, w2[...], preferred_element_type=jnp.float32)
        el2 = jnp.sum(h2 * al[0:1, :], axis=1, keepdims=True)
        er2 = jnp.sum(h2 * ar[0:1, :], axis=1, keepdims=True)
        h2o[...] = h2
        eb2o[...] = jnp.broadcast_to(el2, (_NBK, 16))
        rb2o[...] = jnp.broadcast_to(er2, (_NBK, 16))

    grid = _N // _NBK
    out16 = jax.ShapeDtypeStruct((_N, 16), jnp.float32)
    nspec = pl.BlockSpec((_NBK, 16), lambda i: (i, 0))
    return pl.pallas_call(
        body,
        grid=grid,
        in_specs=[
            pl.BlockSpec((6, _NBK, 16), lambda i: (0, i, 0)),
            pl.BlockSpec((2, _NBK, 16), lambda i: (0, i, 0)),
            _full_spec((48, 16)), _full_spec((8, 48)),
            _full_spec((8, 16)), _full_spec((8, 16)),
        ],
        out_specs=[nspec] * 3,
        out_shape=[out16] * 3,
    )(P, D, W2, b1p, al2p, ar2p)


def _tc3(P2, D2, b2p):
    def body(pref, dref, b2r, outr):
        p = pref[...]
        d = dref[...]
        dh = d[0] + d[1]
        dh = jnp.where(dh == 0.0, 1.0, dh)
        o = (p[0] + p[1]) / dh + b2r[0:1, :]
        outr[...] = jnp.where(o > 0.0, o, 0.01 * o)

    grid = _N // _NBK
    return pl.pallas_call(
        body,
        grid=grid,
        in_specs=[
            pl.BlockSpec((2, _NBK, 16), lambda i: (0, i, 0)),
            pl.BlockSpec((2, _NBK, 16), lambda i: (0, i, 0)),
            _full_spec((8, 16)),
        ],
        out_specs=pl.BlockSpec((_NBK, 16), lambda i: (i, 0)),
        out_shape=jax.ShapeDtypeStruct((_N, 16), jnp.float32),
    )(P2, D2, b2p)


def kernel(inputs, edge_index, emb0, emb1, emb2, W1, al1, ar1, b1,
           W2, al2, ar2, b2):
    src = edge_index[0]
    dst = edge_index[1]

    emb0p = jnp.zeros((16, 8), jnp.float32).at[:14].set(emb0)
    emb1p = jnp.zeros((8, 3), jnp.float32).at[:5].set(emb1)
    emb2p = jnp.zeros((16, 6), jnp.float32).at[:10].set(emb2)
    al1f = jnp.zeros((8, 48), jnp.float32).at[0].set(al1.reshape(48))
    ar1f = jnp.zeros((8, 48), jnp.float32).at[0].set(ar1.reshape(48))
    b1p = jnp.zeros((8, 48), jnp.float32).at[0].set(b1)
    al2p = jnp.zeros((8, 16), jnp.float32).at[0].set(al2.reshape(16))
    ar2p = jnp.zeros((8, 16), jnp.float32).at[0].set(ar2.reshape(16))
    b2p = jnp.zeros((8, 16), jnp.float32).at[0].set(b2)
    zeros = jnp.zeros((_NA, 16), jnp.float32)

    (hh0, hh1, hh2, eb0, eb1, eb2, rb0, rb1, rb2, elt, ert) = _tc1(
        inputs, emb0p, emb1p, emb2p, W1, al1f, ar1f)

    cvec1 = (jnp.maximum(jnp.max(elt, axis=0), 0.0) +
             jnp.maximum(jnp.max(ert, axis=0), 0.0))

    D1 = _edge_denom(src, dst, elt, ert, zeros, cvec1)[:, :_N]
    parts = []
    for hd, (ebr, rbr, hr) in enumerate(
            [(eb0, rb0, hh0), (eb1, rb1, hh1), (eb2, rb2, hh2)]):
        ch = jnp.full((16,), cvec1[hd])
        parts.append(_edge_msg(src, dst, ebr, rbr, hr, zeros, ch)[:, :_N])
    P = jnp.concatenate(parts, axis=0)

    h2, eb2t, rb2t = _tc2(P, D1, W2, b1p, al2p, ar2p)

    c2 = jnp.maximum(jnp.max(eb2t) + jnp.max(rb2t), 0.0)
    c2v = jnp.full((16,), c2)
    D2 = _edge_denom(src, dst, eb2t, rb2t, zeros, c2v)[:, :_N]
    P2 = _edge_msg(src, dst, eb2t, rb2t, h2, zeros, c2v)[:, :_N]

    return _tc3(P2, D2, b2p)


# speedup vs baseline: 34.6081x; 1.3036x over previous
import functools

import jax
import jax.numpy as jnp
from jax import lax
from jax.experimental import pallas as pl
from jax.experimental.pallas import tpu as pltpu
from jax.experimental.pallas import tpu_sc as plsc

_N = 100000
_E = 1600000
_NBK = 1000
_NC = 2
_NS = 16
_NW = _NC * _NS
_CB = 128
_EPW = _E // _NW
_NFULL = _EPW // _CB
_REM = _EPW - _NFULL * _CB
_NA = 100096
_ZST = _NA // _NS


def _make_edge_pass(with_msg):
    mesh = plsc.VectorSubcoreMesh(
        core_axis_name="c", subcore_axis_name="s", num_cores=_NC)

    scratch = [
        pltpu.VMEM((_CB,), jnp.int32),
        pltpu.VMEM((_CB,), jnp.int32),
        pltpu.VMEM((_CB, 16), jnp.float32),
        pltpu.VMEM((_CB, 16), jnp.float32),
        pltpu.VMEM((_CB, 16), jnp.float32),
        pltpu.VMEM((16,), jnp.float32),
        pltpu.VMEM_SHARED((_NA, 16), jnp.float32),
        pltpu.SemaphoreType.DMA,
    ]
    if with_msg:
        scratch.insert(4, pltpu.VMEM((_CB, 16), jnp.float32))

    @functools.partial(
        pl.kernel, mesh=mesh,
        out_type=jax.ShapeDtypeStruct((_NC, _NA, 16), jnp.float32),
        scratch_types=scratch,
        compiler_params=pltpu.CompilerParams(use_tc_tiling_on_sc=False))
    def k(*refs):
        if with_msg:
            (src_h, dst_h, tabA_h, tabB_h, tabH_h, zeros_h, c_h, out_h,
             idxs, idxd, rA, rB, rH, msg, cv, accum, sem) = refs
        else:
            (src_h, dst_h, tabA_h, tabB_h, zeros_h, c_h, out_h,
             idxs, idxd, rA, rB, msg, cv, accum, sem) = refs

        cid = lax.axis_index("c")
        sid = lax.axis_index("s")
        wid = sid * _NC + cid

        pltpu.sync_copy(zeros_h.at[pl.ds(sid * _ZST, _ZST)],
                        accum.at[pl.ds(sid * _ZST, _ZST)])
        pltpu.sync_copy(c_h, cv)
        plsc.subcore_barrier()

        def do_chunk(base, size):
            pltpu.sync_copy(src_h.at[pl.ds(base, size)], idxs.at[pl.ds(0, size)])
            pltpu.sync_copy(dst_h.at[pl.ds(base, size)], idxd.at[pl.ds(0, size)])
            if size < _CB:
                for kk in range(size, _CB, 16):
                    idxs[pl.ds(kk, 16)] = jnp.zeros((16,), jnp.int32)
                    idxd[pl.ds(kk, 16)] = jnp.full((16,), _N, jnp.int32)
            cpA = pltpu.async_copy(tabA_h.at[idxs], rA, sem)
            cpB = pltpu.async_copy(tabB_h.at[idxd], rB, sem)
            if with_msg:
                pltpu.async_copy(tabH_h.at[idxs], rH, sem).wait()
            cpB.wait()
            cpA.wait()
            cvv = cv[...]

            def body(j, carry):
                e = rA[j] + rB[j]
                e = jnp.where(e > 0.0, e, 0.2 * e)
                w = jnp.exp(e - cvv)
                if with_msg:
                    msg[j] = w * rH[j]
                else:
                    msg[j] = w
                return carry

            lax.fori_loop(0, _CB, body, 0)
            pltpu.sync_copy(msg, accum.at[idxd], add=True)

        ebase = wid * _EPW

        def gbody(g, carry):
            do_chunk(ebase + g * _CB, _CB)
            return carry

        lax.fori_loop(0, _NFULL, gbody, 0)
        if _REM:
            do_chunk(ebase + _NFULL * _CB, _REM)

        plsc.subcore_barrier()
        pltpu.sync_copy(accum.at[pl.ds(sid * _ZST, _ZST)],
                        out_h.at[cid, pl.ds(sid * _ZST, _ZST)])

    return k


_edge_denom = _make_edge_pass(with_msg=False)
_edge_msg = _make_edge_pass(with_msg=True)


def _full_spec(shape):
    nd = len(shape)
    return pl.BlockSpec(shape, lambda i: (0,) * nd)


def _tc1(inputs, emb0p, emb1p, emb2p, W1, al1f, ar1f):
    def body(inp, e0, e1, e2, w1, al, ar,
             hh0, hh1, hh2, eb0, eb1, eb2, rb0, rb1, rb2, elt, ert):
        x = inp[...]
        oh0 = (x[:, 0:1].astype(jnp.int32) == lax.broadcasted_iota(
            jnp.int32, (_NBK, 16), 1)).astype(jnp.float32)
        oh1 = (x[:, 1:2].astype(jnp.int32) == lax.broadcasted_iota(
            jnp.int32, (_NBK, 8), 1)).astype(jnp.float32)
        oh2 = (x[:, 2:3].astype(jnp.int32) == lax.broadcasted_iota(
            jnp.int32, (_NBK, 16), 1)).astype(jnp.float32)
        x0 = jnp.dot(oh0, e0[...], preferred_element_type=jnp.float32)
        x1 = jnp.dot(oh1, e1[...], preferred_element_type=jnp.float32)
        x2 = jnp.dot(oh2, e2[...], preferred_element_type=jnp.float32)
        xx = jnp.concatenate([x0, x1, x2, x[:, 3:]], axis=1)
        h = jnp.dot(xx, w1[...], preferred_element_type=jnp.float32)
        sel = (lax.broadcasted_iota(jnp.int32, (48, 3), 0) // 16 ==
               lax.broadcasted_iota(jnp.int32, (48, 3), 1)).astype(jnp.float32)
        el = jnp.dot(h * al[0:1, :], sel,
                     preferred_element_type=jnp.float32)
        er = jnp.dot(h * ar[0:1, :], sel,
                     preferred_element_type=jnp.float32)
        for hd, (hr, ebr, rbr) in enumerate(
                [(hh0, eb0, rb0), (hh1, eb1, rb1), (hh2, eb2, rb2)]):
            hr[...] = h[:, hd * 16:(hd + 1) * 16]
            ebr[...] = jnp.broadcast_to(el[:, hd:hd + 1], (_NBK, 16))
            rbr[...] = jnp.broadcast_to(er[:, hd:hd + 1], (_NBK, 16))
        z = jnp.zeros((_NBK, 13), jnp.float32)
        elt[...] = jnp.concatenate([el, z], axis=1)
        ert[...] = jnp.concatenate([er, z], axis=1)

    grid = _N // _NBK
    out16 = jax.ShapeDtypeStruct((_N, 16), jnp.float32)
    nspec = pl.BlockSpec((_NBK, 16), lambda i: (i, 0))
    return pl.pallas_call(
        body,
        grid=grid,
        in_specs=[
            pl.BlockSpec((_NBK, 18), lambda i: (i, 0)),
            _full_spec((16, 8)), _full_spec((8, 3)), _full_spec((16, 6)),
            _full_spec((32, 48)), _full_spec((8, 48)), _full_spec((8, 48)),
        ],
        out_specs=[nspec] * 11,
        out_shape=[out16] * 11,
    )(inputs, emb0p, emb1p, emb2p, W1, al1f, ar1f)


def _tc2(P, D, W2, b1p, al2p, ar2p):
    def body(pref, dref, w2, b1r, al, ar, h2o, eb2o, rb2o):
        p = pref[...]
        d = dref[...]
        dh = d[0] + d[1]
        dh = jnp.where(dh == 0.0, 1.0, dh)
        cols = []
        for hd in range(3):
            num = p[2 * hd] + p[2 * hd + 1]
            cols.append(num / dh[:, hd:hd + 1])
        hid = jnp.concatenate(cols, axis=1) + b1r[0:1, :]
        hid = jnp.where(hid > 0.0, hid, 0.01 * hid)
        h2 = jnp.dot(hid, w2[...], preferred_element_type=jnp.float32)
        el2 = jnp.sum(h2 * al[0:1, :], axis=1, keepdims=True)
        er2 = jnp.sum(h2 * ar[0:1, :], axis=1, keepdims=True)
        h2o[...] = h2
        eb2o[...] = jnp.broadcast_to(el2, (_NBK, 16))
        rb2o[...] = jnp.broadcast_to(er2, (_NBK, 16))

    grid = _N // _NBK
    out16 = jax.ShapeDtypeStruct((_N, 16), jnp.float32)
    nspec = pl.BlockSpec((_NBK, 16), lambda i: (i, 0))
    return pl.pallas_call(
        body,
        grid=grid,
        in_specs=[
            pl.BlockSpec((6, _NBK, 16), lambda i: (0, i, 0)),
            pl.BlockSpec((2, _NBK, 16), lambda i: (0, i, 0)),
            _full_spec((48, 16)), _full_spec((8, 48)),
            _full_spec((8, 16)), _full_spec((8, 16)),
        ],
        out_specs=[nspec] * 3,
        out_shape=[out16] * 3,
    )(P, D, W2, b1p, al2p, ar2p)


def _tc3(P2, D2, b2p):
    def body(pref, dref, b2r, outr):
        p = pref[...]
        d = dref[...]
        dh = d[0] + d[1]
        dh = jnp.where(dh == 0.0, 1.0, dh)
        o = (p[0] + p[1]) / dh + b2r[0:1, :]
        outr[...] = jnp.where(o > 0.0, o, 0.01 * o)

    grid = _N // _NBK
    return pl.pallas_call(
        body,
        grid=grid,
        in_specs=[
            pl.BlockSpec((2, _NBK, 16), lambda i: (0, i, 0)),
            pl.BlockSpec((2, _NBK, 16), lambda i: (0, i, 0)),
            _full_spec((8, 16)),
        ],
        out_specs=pl.BlockSpec((_NBK, 16), lambda i: (i, 0)),
        out_shape=jax.ShapeDtypeStruct((_N, 16), jnp.float32),
    )(P2, D2, b2p)


def kernel(inputs, edge_index, emb0, emb1, emb2, W1, al1, ar1, b1,
           W2, al2, ar2, b2):
    src = edge_index[0]
    dst = edge_index[1]

    emb0p = jnp.zeros((16, 8), jnp.float32).at[:14].set(emb0)
    emb1p = jnp.zeros((8, 3), jnp.float32).at[:5].set(emb1)
    emb2p = jnp.zeros((16, 6), jnp.float32).at[:10].set(emb2)
    al1f = jnp.zeros((8, 48), jnp.float32).at[0].set(al1.reshape(48))
    ar1f = jnp.zeros((8, 48), jnp.float32).at[0].set(ar1.reshape(48))
    b1p = jnp.zeros((8, 48), jnp.float32).at[0].set(b1)
    al2p = jnp.zeros((8, 16), jnp.float32).at[0].set(al2.reshape(16))
    ar2p = jnp.zeros((8, 16), jnp.float32).at[0].set(ar2.reshape(16))
    b2p = jnp.zeros((8, 16), jnp.float32).at[0].set(b2)
    zeros = jnp.zeros((_NA, 16), jnp.float32)

    (hh0, hh1, hh2, eb0, eb1, eb2, rb0, rb1, rb2, elt, ert) = _tc1(
        inputs, emb0p, emb1p, emb2p, W1, al1f, ar1f)

    cvec1 = (jnp.maximum(jnp.max(elt, axis=0), 0.0) +
             jnp.maximum(jnp.max(ert, axis=0), 0.0))

    D1 = _edge_denom(src, dst, elt, ert, zeros, cvec1)[:, :_N]
    parts = []
    for hd, (ebr, rbr, hr) in enumerate(
            [(eb0, rb0, hh0), (eb1, rb1, hh1), (eb2, rb2, hh2)]):
        ch = jnp.full((16,), cvec1[hd])
        parts.append(_edge_msg(src, dst, ebr, rbr, hr, zeros, ch)[:, :_N])
    P = jnp.concatenate(parts, axis=0)

    h2, eb2t, rb2t = _tc2(P, D1, W2, b1p, al2p, ar2p)

    c2 = jnp.maximum(jnp.max(eb2t) + jnp.max(rb2t), 0.0)
    c2v = jnp.full((16,), c2)
    D2 = _edge_denom(src, dst, eb2t, rb2t, zeros, c2v)[:, :_N]
    P2 = _edge_msg(src, dst, eb2t, rb2t, h2, zeros, c2v)[:, :_N]

    return _tc3(P2, D2, b2p)
